# baseline (speedup 1.0000x reference)
import functools

import jax
import jax.numpy as jnp
from jax import lax
from jax.experimental import pallas as pl
from jax.experimental.pallas import tpu as pltpu
from jax.experimental.pallas import tpu_sc as plsc

NC = 2
NS = 16
NW = NC * NS
CH = 125
NB = 8
GA = 4
RCH = 80
HD = 128
DEGW = 16
NUM_GRAPHS = 16


def _sc_mesh():
    return plsc.VectorSubcoreMesh(
        core_axis_name="c", subcore_axis_name="s", num_cores=NC, num_subcores=NS
    )


@functools.lru_cache(maxsize=None)
def _edge_scatter_kernel(n_nodes, n_chunks):
    n_zchunks = n_nodes // RCH
    assert n_zchunks * RCH == n_nodes

    @functools.partial(
        pl.kernel,
        out_type=jax.ShapeDtypeStruct((NC, n_nodes, HD), jnp.bfloat16),
        mesh=_sc_mesh(),
        compiler_params=pltpu.CompilerParams(use_tc_tiling_on_sc=False),
        scratch_types=[
            pltpu.VMEM((n_chunks, CH), jnp.int32),
            pltpu.VMEM((n_chunks, CH), jnp.int32),
            pltpu.VMEM((NB, CH, HD), jnp.bfloat16),
            pltpu.SemaphoreType.DMA((NB,)),
            pltpu.SemaphoreType.DMA((NB,)),
            pltpu.VMEM_SHARED((n_nodes, HD), jnp.bfloat16),
        ],
    )
    def k(u_hbm, src_hbm, dst_hbm, out_hbm, srcb, dstb, rows, gsems, ssems,
          acc):
        cid = lax.axis_index("c")
        sid = lax.axis_index("s")
        wid = cid * NS + sid

        @pl.loop(0, CH)
        def _(r):
            for j in range(HD // 32):
                rows[0, r, pl.ds(j * 32, 32)] = jnp.zeros((32,), jnp.bfloat16)

        for j in range(pl.cdiv(n_zchunks, NS)):
            kk = sid + NS * j
            @pl.when(kk < n_zchunks)
            def _():
                pltpu.sync_copy(rows.at[0, pl.ds(0, RCH)],
                                acc.at[pl.ds(kk * RCH, RCH)])
        plsc.subcore_barrier()

        pltpu.sync_copy(src_hbm.at[wid], srcb)
        pltpu.sync_copy(dst_hbm.at[wid], dstb)

        def gather(c, j):
            pltpu.async_copy(u_hbm.at[srcb.at[c]], rows.at[j], gsems.at[j])

        def gwait(j):
            pltpu.make_async_copy(u_hbm.at[srcb.at[0]], rows.at[j],
                                  gsems.at[j]).wait()

        def scat(c, j):
            pltpu.async_copy(rows.at[j], acc.at[dstb.at[c]], ssems.at[j],
                             add=True)

        def swait(j):
            pltpu.make_async_copy(rows.at[j], acc.at[dstb.at[0]],
                                  ssems.at[j]).wait()

        for c in range(GA):
            gather(c, c)

        @pl.loop(0, n_chunks)
        def _(i):
            m = i + GA
            @pl.when(m < n_chunks)
            def _():
                jm = lax.rem(m, NB)
                @pl.when(i >= GA)
                def _():
                    swait(jm)
                gather(m, jm)
            ji = lax.rem(i, NB)
            gwait(ji)
            scat(i, ji)

        @pl.loop(max(n_chunks - GA, 0), n_chunks)
        def _(i):
            swait(lax.rem(i, NB))
        plsc.subcore_barrier()

        for j in range(pl.cdiv(n_zchunks, NS)):
            kk = sid + NS * j
            @pl.when(kk < n_zchunks)
            def _():
                pltpu.sync_copy(acc.at[pl.ds(kk * RCH, RCH)],
                                out_hbm.at[cid, pl.ds(kk * RCH, RCH)])

    return k


@functools.lru_cache(maxsize=None)
def _degree_kernel(n_nodes, n_chunks):
    n_rchunks = n_nodes // RCH
    assert n_rchunks * RCH == n_nodes

    @functools.partial(
        pl.kernel,
        out_type=jax.ShapeDtypeStruct((NC, n_nodes, DEGW), jnp.float32),
        mesh=_sc_mesh(),
        compiler_params=pltpu.CompilerParams(use_tc_tiling_on_sc=False),
        scratch_types=[
            pltpu.VMEM((n_chunks, CH), jnp.int32),
            pltpu.VMEM((CH, DEGW), jnp.float32),
            pltpu.VMEM((CH, DEGW), jnp.float32),
            pltpu.SemaphoreType.DMA,
            pltpu.VMEM_SHARED((n_nodes, DEGW), jnp.float32),
        ],
    )
    def k(dst_hbm, out_hbm, dstb, zeros_v, ones_v, sem, acc):
        cid = lax.axis_index("c")
        sid = lax.axis_index("s")
        wid = cid * NS + sid

        @pl.loop(0, CH)
        def _(r):
            zeros_v[r, :] = jnp.zeros((DEGW,), jnp.float32)
            ones_v[r, :] = jnp.ones((DEGW,), jnp.float32)

        for j in range(pl.cdiv(n_rchunks, NS)):
            kk = sid + NS * j
            @pl.when(kk < n_rchunks)
            def _():
                pltpu.sync_copy(zeros_v.at[pl.ds(0, RCH)],
                                acc.at[pl.ds(kk * RCH, RCH)])
        plsc.subcore_barrier()

        pltpu.sync_copy(dst_hbm.at[wid], dstb)

        @pl.loop(0, n_chunks)
        def _(c):
            pltpu.async_copy(ones_v, acc.at[dstb.at[c]], sem, add=True)

        @pl.loop(0, n_chunks)
        def _(c):
            pltpu.make_async_copy(ones_v, acc.at[dstb.at[0]], sem).wait()

        plsc.subcore_barrier()

        for j in range(pl.cdiv(n_rchunks, NS)):
            kk = sid + NS * j
            @pl.when(kk < n_rchunks)
            def _():
                pltpu.sync_copy(acc.at[pl.ds(kk * RCH, RCH)],
                                out_hbm.at[cid, pl.ds(kk * RCH, RCH)])

    return k


def _tc_matmul(x, w):
    n = x.shape[0]

    def body(x_ref, w_ref, h_ref):
        h_ref[...] = jnp.dot(x_ref[...], w_ref[...],
                             preferred_element_type=jnp.float32)

    return pl.pallas_call(
        body, out_shape=jax.ShapeDtypeStruct((n, w.shape[1]), jnp.float32),
    )(x, w)


def _tc_scale(degp, h0):
    n = h0.shape[0]

    def body(degp_ref, h_ref, dis_ref, u_ref, ubf_ref):
        deg = degp_ref[0, :, 0:1] + degp_ref[1, :, 0:1] + 1.0
        dis = lax.rsqrt(deg)
        dis_ref[...] = dis
        u = dis * h_ref[...]
        u_ref[...] = u
        ubf_ref[...] = u.astype(jnp.bfloat16)

    return pl.pallas_call(
        body,
        out_shape=(
            jax.ShapeDtypeStruct((n, 1), jnp.float32),
            jax.ShapeDtypeStruct((n, HD), jnp.float32),
            jax.ShapeDtypeStruct((n, HD), jnp.bfloat16),
        ),
    )(degp, h0)


def _tc_mid(p, u, dis, b, w):
    n = u.shape[0]

    def body(p_ref, u_ref, dis_ref, b_ref, w_ref, un_ref, unbf_ref):
        s = (p_ref[0].astype(jnp.float32) + p_ref[1].astype(jnp.float32)
             + u_ref[...])
        h = jnp.maximum(dis_ref[...] * s + b_ref[...], 0.0)
        un = dis_ref[...] * jnp.dot(h, w_ref[...],
                                    preferred_element_type=jnp.float32)
        un_ref[...] = un
        unbf_ref[...] = un.astype(jnp.bfloat16)

    return pl.pallas_call(
        body,
        out_shape=(
            jax.ShapeDtypeStruct((n, HD), jnp.float32),
            jax.ShapeDtypeStruct((n, HD), jnp.bfloat16),
        ),
    )(p, u, dis, b.reshape(1, -1), w)


def _tc_last(p, u, dis, b, batch, lin_w, lin_b):
    n = u.shape[0]
    ncls = lin_w.shape[1]

    def body(p_ref, u_ref, dis_ref, b_ref, batch_ref, lw_ref, lb_ref, out_ref):
        s = (p_ref[0].astype(jnp.float32) + p_ref[1].astype(jnp.float32)
             + u_ref[...])
        h = jnp.maximum(dis_ref[...] * s + b_ref[...], 0.0)
        oh = (batch_ref[...] ==
              lax.broadcasted_iota(jnp.int32, (n, NUM_GRAPHS), 1)
              ).astype(jnp.float32)
        sums = lax.dot_general(oh, h, (((0,), (0,)), ((), ())),
                               preferred_element_type=jnp.float32)
        counts = jnp.sum(oh, axis=0)
        pooled = sums / jnp.maximum(counts, 1.0)[:, None]
        out_ref[...] = jnp.dot(pooled, lw_ref[...],
                               preferred_element_type=jnp.float32) + lb_ref[...]

    return pl.pallas_call(
        body,
        out_shape=jax.ShapeDtypeStruct((NUM_GRAPHS, ncls), jnp.float32),
    )(p, u, dis, b.reshape(1, -1), batch.reshape(-1, 1), lin_w, lin_b.reshape(1, -1))


def kernel(x, edge_index, batch, W0, b0, W1, b1, W2, b2, W3, b3, W4, b4, lin_W, lin_b):
    n, d = x.shape
    e = edge_index.shape[1]
    e_per_w = e // NW
    n_chunks = e_per_w // CH
    assert n_chunks * CH == e_per_w
    src32 = edge_index[0].reshape(NW, n_chunks, CH)
    dst32 = edge_index[1].reshape(NW, n_chunks, CH)

    degp = _degree_kernel(n, n_chunks)(dst32)
    h0 = _tc_matmul(x, W0)
    dis, u, ubf = _tc_scale(degp, h0)

    scatter = _edge_scatter_kernel(n, n_chunks)
    for w_next, b_cur in ((W1, b0), (W2, b1), (W3, b2), (W4, b3)):
        p = scatter(ubf, src32, dst32)
        u, ubf = _tc_mid(p, u, dis, b_cur, w_next)

    p = scatter(ubf, src32, dst32)
    return _tc_last(p, u, dis, b4, batch, lin_W, lin_b)


# speedup vs baseline: 1.0529x; 1.0529x over previous
import functools

import jax
import jax.numpy as jnp
from jax import lax
from jax.experimental import pallas as pl
from jax.experimental.pallas import tpu as pltpu
from jax.experimental.pallas import tpu_sc as plsc

NC = 2
NS = 16
NW = NC * NS
CH = 125
NB = 6
GA = 3
RCH = 80
HD = 128
DEGW = 16
NUM_GRAPHS = 16


def _sc_mesh():
    return plsc.VectorSubcoreMesh(
        core_axis_name="c", subcore_axis_name="s", num_cores=NC, num_subcores=NS
    )


@functools.lru_cache(maxsize=None)
def _edge_scatter_kernel(n_nodes, n_chunks):
    n_zchunks = n_nodes // RCH
    assert n_zchunks * RCH == n_nodes

    @functools.partial(
        pl.kernel,
        out_type=jax.ShapeDtypeStruct((NC, n_nodes, HD), jnp.bfloat16),
        mesh=_sc_mesh(),
        compiler_params=pltpu.CompilerParams(use_tc_tiling_on_sc=False),
        scratch_types=[
            pltpu.VMEM((n_chunks, CH), jnp.int32),
            pltpu.VMEM((n_chunks, CH), jnp.int32),
            pltpu.VMEM((NB, CH, HD), jnp.bfloat16),
            pltpu.SemaphoreType.DMA((NB,)),
            pltpu.SemaphoreType.DMA((NB,)),
            pltpu.VMEM_SHARED((n_nodes, HD), jnp.bfloat16),
        ],
    )
    def k(u_hbm, src_hbm, dst_hbm, out_hbm, srcb, dstb, rows, gsems, ssems,
          acc):
        cid = lax.axis_index("c")
        sid = lax.axis_index("s")
        wid = cid * NS + sid

        pltpu.async_copy(src_hbm.at[wid], srcb, gsems.at[NB - 1])
        pltpu.async_copy(dst_hbm.at[wid], dstb, ssems.at[NB - 1])

        @pl.loop(0, CH)
        def _(r):
            for j in range(HD // 32):
                rows[GA, r, pl.ds(j * 32, 32)] = jnp.zeros((32,), jnp.bfloat16)

        pltpu.make_async_copy(src_hbm.at[wid], srcb, gsems.at[NB - 1]).wait()

        def gather(c, j):
            pltpu.async_copy(u_hbm.at[srcb.at[c]], rows.at[j], gsems.at[j])

        def gwait(j):
            pltpu.make_async_copy(u_hbm.at[srcb.at[0]], rows.at[j],
                                  gsems.at[j]).wait()

        def scat(c, j):
            pltpu.async_copy(rows.at[j], acc.at[dstb.at[c]], ssems.at[j],
                             add=True)

        def swait(j):
            pltpu.make_async_copy(rows.at[j], acc.at[dstb.at[0]],
                                  ssems.at[j]).wait()

        for c in range(GA):
            gather(c, c)

        for j in range(pl.cdiv(n_zchunks, NS)):
            kk = sid + NS * j
            @pl.when(kk < n_zchunks)
            def _():
                pltpu.async_copy(rows.at[GA, pl.ds(0, RCH)],
                                 acc.at[pl.ds(kk * RCH, RCH)], ssems.at[GA])
        for j in range(pl.cdiv(n_zchunks, NS)):
            kk = sid + NS * j
            @pl.when(kk < n_zchunks)
            def _():
                pltpu.make_async_copy(rows.at[GA, pl.ds(0, RCH)],
                                      acc.at[pl.ds(0, RCH)], ssems.at[GA]).wait()

        pltpu.make_async_copy(dst_hbm.at[wid], dstb, ssems.at[NB - 1]).wait()
        plsc.subcore_barrier()

        @pl.loop(0, n_chunks)
        def _(i):
            m = i + GA
            @pl.when(m < n_chunks)
            def _():
                jm = lax.rem(m, NB)
                @pl.when(i >= GA)
                def _():
                    swait(jm)
                gather(m, jm)
            ji = lax.rem(i, NB)
            gwait(ji)
            scat(i, ji)

        @pl.loop(max(n_chunks - GA, 0), n_chunks)
        def _(i):
            swait(lax.rem(i, NB))
        plsc.subcore_barrier()

        for j in range(pl.cdiv(n_zchunks, NS)):
            kk = sid + NS * j
            @pl.when(kk < n_zchunks)
            def _():
                pltpu.async_copy(acc.at[pl.ds(kk * RCH, RCH)],
                                 out_hbm.at[cid, pl.ds(kk * RCH, RCH)],
                                 gsems.at[0])
        for j in range(pl.cdiv(n_zchunks, NS)):
            kk = sid + NS * j
            @pl.when(kk < n_zchunks)
            def _():
                pltpu.make_async_copy(acc.at[pl.ds(0, RCH)],
                                      out_hbm.at[cid, pl.ds(0, RCH)],
                                      gsems.at[0]).wait()

    return k


@functools.lru_cache(maxsize=None)
def _degree_kernel(n_nodes, n_chunks):
    n_rchunks = n_nodes // RCH
    assert n_rchunks * RCH == n_nodes

    @functools.partial(
        pl.kernel,
        out_type=jax.ShapeDtypeStruct((NC, n_nodes, DEGW), jnp.float32),
        mesh=_sc_mesh(),
        compiler_params=pltpu.CompilerParams(use_tc_tiling_on_sc=False),
        scratch_types=[
            pltpu.VMEM((n_chunks, CH), jnp.int32),
            pltpu.VMEM((CH, DEGW), jnp.float32),
            pltpu.VMEM((CH, DEGW), jnp.float32),
            pltpu.SemaphoreType.DMA,
            pltpu.VMEM_SHARED((n_nodes, DEGW), jnp.float32),
        ],
    )
    def k(dst_hbm, out_hbm, dstb, zeros_v, ones_v, sem, acc):
        cid = lax.axis_index("c")
        sid = lax.axis_index("s")
        wid = cid * NS + sid

        @pl.loop(0, CH)
        def _(r):
            zeros_v[r, :] = jnp.zeros((DEGW,), jnp.float32)
            ones_v[r, :] = jnp.ones((DEGW,), jnp.float32)

        for j in range(pl.cdiv(n_rchunks, NS)):
            kk = sid + NS * j
            @pl.when(kk < n_rchunks)
            def _():
                pltpu.sync_copy(zeros_v.at[pl.ds(0, RCH)],
                                acc.at[pl.ds(kk * RCH, RCH)])
        plsc.subcore_barrier()

        pltpu.sync_copy(dst_hbm.at[wid], dstb)

        @pl.loop(0, n_chunks)
        def _(c):
            pltpu.async_copy(ones_v, acc.at[dstb.at[c]], sem, add=True)

        @pl.loop(0, n_chunks)
        def _(c):
            pltpu.make_async_copy(ones_v, acc.at[dstb.at[0]], sem).wait()

        plsc.subcore_barrier()

        for j in range(pl.cdiv(n_rchunks, NS)):
            kk = sid + NS * j
            @pl.when(kk < n_rchunks)
            def _():
                pltpu.sync_copy(acc.at[pl.ds(kk * RCH, RCH)],
                                out_hbm.at[cid, pl.ds(kk * RCH, RCH)])

    return k


def _tc_matmul(x, w):
    n = x.shape[0]

    def body(x_ref, w_ref, h_ref):
        h_ref[...] = jnp.dot(x_ref[...], w_ref[...],
                             preferred_element_type=jnp.float32)

    return pl.pallas_call(
        body, out_shape=jax.ShapeDtypeStruct((n, w.shape[1]), jnp.float32),
    )(x, w)


def _tc_scale(degp, h0):
    n = h0.shape[0]

    def body(degp_ref, h_ref, dis_ref, u_ref, ubf_ref):
        deg = degp_ref[0, :, 0:1] + degp_ref[1, :, 0:1] + 1.0
        dis = lax.rsqrt(deg)
        dis_ref[...] = dis
        u = dis * h_ref[...]
        u_ref[...] = u
        ubf_ref[...] = u.astype(jnp.bfloat16)

    return pl.pallas_call(
        body,
        out_shape=(
            jax.ShapeDtypeStruct((n, 1), jnp.float32),
            jax.ShapeDtypeStruct((n, HD), jnp.float32),
            jax.ShapeDtypeStruct((n, HD), jnp.bfloat16),
        ),
    )(degp, h0)


def _tc_mid(p, u, dis, b, w):
    n = u.shape[0]

    def body(p_ref, u_ref, dis_ref, b_ref, w_ref, un_ref, unbf_ref):
        s = (p_ref[0].astype(jnp.float32) + p_ref[1].astype(jnp.float32)
             + u_ref[...])
        h = jnp.maximum(dis_ref[...] * s + b_ref[...], 0.0)
        un = dis_ref[...] * jnp.dot(h, w_ref[...],
                                    preferred_element_type=jnp.float32)
        un_ref[...] = un
        unbf_ref[...] = un.astype(jnp.bfloat16)

    return pl.pallas_call(
        body,
        out_shape=(
            jax.ShapeDtypeStruct((n, HD), jnp.float32),
            jax.ShapeDtypeStruct((n, HD), jnp.bfloat16),
        ),
    )(p, u, dis, b.reshape(1, -1), w)


def _tc_last(p, u, dis, b, batch, lin_w, lin_b):
    n = u.shape[0]
    ncls = lin_w.shape[1]

    def body(p_ref, u_ref, dis_ref, b_ref, batch_ref, lw_ref, lb_ref, out_ref):
        s = (p_ref[0].astype(jnp.float32) + p_ref[1].astype(jnp.float32)
             + u_ref[...])
        h = jnp.maximum(dis_ref[...] * s + b_ref[...], 0.0)
        oh = (batch_ref[...] ==
              lax.broadcasted_iota(jnp.int32, (n, NUM_GRAPHS), 1)
              ).astype(jnp.float32)
        sums = lax.dot_general(oh, h, (((0,), (0,)), ((), ())),
                               preferred_element_type=jnp.float32)
        counts = jnp.sum(oh, axis=0)
        pooled = sums / jnp.maximum(counts, 1.0)[:, None]
        out_ref[...] = jnp.dot(pooled, lw_ref[...],
                               preferred_element_type=jnp.float32) + lb_ref[...]

    return pl.pallas_call(
        body,
        out_shape=jax.ShapeDtypeStruct((NUM_GRAPHS, ncls), jnp.float32),
    )(p, u, dis, b.reshape(1, -1), batch.reshape(-1, 1), lin_w, lin_b.reshape(1, -1))


def kernel(x, edge_index, batch, W0, b0, W1, b1, W2, b2, W3, b3, W4, b4, lin_W, lin_b):
    n, d = x.shape
    e = edge_index.shape[1]
    e_per_w = e // NW
    n_chunks = e_per_w // CH
    assert n_chunks * CH == e_per_w
    src32 = edge_index[0].reshape(NW, n_chunks, CH)
    dst32 = edge_index[1].reshape(NW, n_chunks, CH)

    degp = _degree_kernel(n, n_chunks)(dst32)
    h0 = _tc_matmul(x, W0)
    dis, u, ubf = _tc_scale(degp, h0)

    scatter = _edge_scatter_kernel(n, n_chunks)
    for w_next, b_cur in ((W1, b0), (W2, b1), (W3, b2), (W4, b3)):
        p = scatter(ubf, src32, dst32)
        u, ubf = _tc_mid(p, u, dis, b_cur, w_next)

    p = scatter(ubf, src32, dst32)
    return _tc_last(p, u, dis, b4, batch, lin_W, lin_b)


# speedup vs baseline: 1.0719x; 1.0181x over previous
import functools

import jax
import jax.numpy as jnp
from jax import lax
from jax.experimental import pallas as pl
from jax.experimental.pallas import tpu as pltpu
from jax.experimental.pallas import tpu_sc as plsc

NC = 2
NS = 16
NW = NC * NS
CH = 125
NB = 6
GA = 3
RCH = 80
HD = 128
DEGW = 16
NUM_GRAPHS = 16


def _sc_mesh():
    return plsc.VectorSubcoreMesh(
        core_axis_name="c", subcore_axis_name="s", num_cores=NC, num_subcores=NS
    )


@functools.lru_cache(maxsize=None)
def _edge_scatter_kernel(n_nodes, n_chunks):
    n_zchunks = n_nodes // RCH
    assert n_zchunks * RCH == n_nodes

    @functools.partial(
        pl.kernel,
        out_type=jax.ShapeDtypeStruct((NC, n_nodes, HD), jnp.bfloat16),
        mesh=_sc_mesh(),
        compiler_params=pltpu.CompilerParams(use_tc_tiling_on_sc=False),
        scratch_types=[
            pltpu.VMEM((n_chunks, CH), jnp.int32),
            pltpu.VMEM((n_chunks, CH), jnp.int32),
            pltpu.VMEM((NB, CH, HD), jnp.bfloat16),
            pltpu.SemaphoreType.DMA((NB,)),
            pltpu.SemaphoreType.DMA((NB,)),
            pltpu.VMEM_SHARED((n_nodes, HD), jnp.bfloat16),
        ],
    )
    def k(u_hbm, src_hbm, dst_hbm, out_hbm, srcb, dstb, rows, gsems, ssems,
          acc):
        cid = lax.axis_index("c")
        sid = lax.axis_index("s")
        wid = cid * NS + sid

        pltpu.async_copy(src_hbm.at[wid], srcb, gsems.at[NB - 1])
        pltpu.async_copy(dst_hbm.at[wid], dstb, ssems.at[NB - 1])

        @pl.loop(0, CH)
        def _(r):
            for j in range(HD // 32):
                rows[GA, r, pl.ds(j * 32, 32)] = jnp.zeros((32,), jnp.bfloat16)

        pltpu.make_async_copy(src_hbm.at[wid], srcb, gsems.at[NB - 1]).wait()

        def gather(c, j):
            pltpu.async_copy(u_hbm.at[srcb.at[c]], rows.at[j], gsems.at[j])

        def gwait(j):
            pltpu.make_async_copy(u_hbm.at[srcb.at[0]], rows.at[j],
                                  gsems.at[j]).wait()

        def scat(c, j):
            pltpu.async_copy(rows.at[j], acc.at[dstb.at[c]], ssems.at[j],
                             add=True)

        def swait(j):
            pltpu.make_async_copy(rows.at[j], acc.at[dstb.at[0]],
                                  ssems.at[j]).wait()

        for c in range(GA):
            gather(c, c)

        for j in range(pl.cdiv(n_zchunks, NS)):
            kk = sid + NS * j
            @pl.when(kk < n_zchunks)
            def _():
                pltpu.async_copy(rows.at[GA, pl.ds(0, RCH)],
                                 acc.at[pl.ds(kk * RCH, RCH)], ssems.at[GA])
        for j in range(pl.cdiv(n_zchunks, NS)):
            kk = sid + NS * j
            @pl.when(kk < n_zchunks)
            def _():
                pltpu.make_async_copy(rows.at[GA, pl.ds(0, RCH)],
                                      acc.at[pl.ds(0, RCH)], ssems.at[GA]).wait()

        pltpu.make_async_copy(dst_hbm.at[wid], dstb, ssems.at[NB - 1]).wait()
        plsc.subcore_barrier()

        @pl.loop(0, n_chunks)
        def _(i):
            m = i + GA
            @pl.when(m < n_chunks)
            def _():
                jm = lax.rem(m, NB)
                @pl.when(i >= GA)
                def _():
                    swait(jm)
                gather(m, jm)
            ji = lax.rem(i, NB)
            gwait(ji)
            scat(i, ji)

        @pl.loop(max(n_chunks - NB, 0), n_chunks)
        def _(i):
            @pl.when(i >= GA + max(n_chunks - NB, 0) - max(n_chunks - GA, 0)
                     if False else True)
            def _():
                pass
            swait(lax.rem(i, NB))
        plsc.subcore_barrier()

        for j in range(pl.cdiv(n_zchunks, NS)):
            kk = sid + NS * j
            @pl.when(kk < n_zchunks)
            def _():
                pltpu.async_copy(acc.at[pl.ds(kk * RCH, RCH)],
                                 out_hbm.at[cid, pl.ds(kk * RCH, RCH)],
                                 gsems.at[0])
        for j in range(pl.cdiv(n_zchunks, NS)):
            kk = sid + NS * j
            @pl.when(kk < n_zchunks)
            def _():
                pltpu.make_async_copy(acc.at[pl.ds(0, RCH)],
                                      out_hbm.at[cid, pl.ds(0, RCH)],
                                      gsems.at[0]).wait()

    return k


@functools.lru_cache(maxsize=None)
def _degree_kernel(n_nodes, n_chunks):
    n_rchunks = n_nodes // RCH
    assert n_rchunks * RCH == n_nodes

    @functools.partial(
        pl.kernel,
        out_type=jax.ShapeDtypeStruct((NC, n_nodes, DEGW), jnp.float32),
        mesh=_sc_mesh(),
        compiler_params=pltpu.CompilerParams(use_tc_tiling_on_sc=False),
        scratch_types=[
            pltpu.VMEM((n_chunks, CH), jnp.int32),
            pltpu.VMEM((CH, DEGW), jnp.float32),
            pltpu.VMEM((CH, DEGW), jnp.float32),
            pltpu.SemaphoreType.DMA,
            pltpu.VMEM_SHARED((n_nodes, DEGW), jnp.float32),
        ],
    )
    def k(dst_hbm, out_hbm, dstb, zeros_v, ones_v, sem, acc):
        cid = lax.axis_index("c")
        sid = lax.axis_index("s")
        wid = cid * NS + sid

        @pl.loop(0, CH)
        def _(r):
            zeros_v[r, :] = jnp.zeros((DEGW,), jnp.float32)
            ones_v[r, :] = jnp.ones((DEGW,), jnp.float32)

        for j in range(pl.cdiv(n_rchunks, NS)):
            kk = sid + NS * j
            @pl.when(kk < n_rchunks)
            def _():
                pltpu.sync_copy(zeros_v.at[pl.ds(0, RCH)],
                                acc.at[pl.ds(kk * RCH, RCH)])
        plsc.subcore_barrier()

        pltpu.sync_copy(dst_hbm.at[wid], dstb)

        @pl.loop(0, n_chunks)
        def _(c):
            pltpu.async_copy(ones_v, acc.at[dstb.at[c]], sem, add=True)

        @pl.loop(0, n_chunks)
        def _(c):
            pltpu.make_async_copy(ones_v, acc.at[dstb.at[0]], sem).wait()

        plsc.subcore_barrier()

        for j in range(pl.cdiv(n_rchunks, NS)):
            kk = sid + NS * j
            @pl.when(kk < n_rchunks)
            def _():
                pltpu.sync_copy(acc.at[pl.ds(kk * RCH, RCH)],
                                out_hbm.at[cid, pl.ds(kk * RCH, RCH)])

    return k


def _tc_matmul(x, w):
    n = x.shape[0]

    def body(x_ref, w_ref, h_ref):
        h_ref[...] = jnp.dot(x_ref[...], w_ref[...],
                             preferred_element_type=jnp.float32)

    return pl.pallas_call(
        body, out_shape=jax.ShapeDtypeStruct((n, w.shape[1]), jnp.float32),
    )(x, w)


def _tc_scale(degp, h0):
    n = h0.shape[0]

    def body(degp_ref, h_ref, dis_ref, u_ref, ubf_ref):
        deg = degp_ref[0, :, 0:1] + degp_ref[1, :, 0:1] + 1.0
        dis = lax.rsqrt(deg)
        dis_ref[...] = dis
        u = dis * h_ref[...]
        u_ref[...] = u
        ubf_ref[...] = u.astype(jnp.bfloat16)

    return pl.pallas_call(
        body,
        out_shape=(
            jax.ShapeDtypeStruct((n, 1), jnp.float32),
            jax.ShapeDtypeStruct((n, HD), jnp.float32),
            jax.ShapeDtypeStruct((n, HD), jnp.bfloat16),
        ),
    )(degp, h0)


def _tc_mid(p, u, dis, b, w):
    n = u.shape[0]

    def body(p_ref, u_ref, dis_ref, b_ref, w_ref, un_ref, unbf_ref):
        s = (p_ref[0].astype(jnp.float32) + p_ref[1].astype(jnp.float32)
             + u_ref[...])
        h = jnp.maximum(dis_ref[...] * s + b_ref[...], 0.0)
        un = dis_ref[...] * jnp.dot(h, w_ref[...],
                                    preferred_element_type=jnp.float32)
        un_ref[...] = un
        unbf_ref[...] = un.astype(jnp.bfloat16)

    return pl.pallas_call(
        body,
        out_shape=(
            jax.ShapeDtypeStruct((n, HD), jnp.float32),
            jax.ShapeDtypeStruct((n, HD), jnp.bfloat16),
        ),
    )(p, u, dis, b.reshape(1, -1), w)


def _tc_last(p, u, dis, b, batch, lin_w, lin_b):
    n = u.shape[0]
    ncls = lin_w.shape[1]

    def body(p_ref, u_ref, dis_ref, b_ref, batch_ref, lw_ref, lb_ref, out_ref):
        s = (p_ref[0].astype(jnp.float32) + p_ref[1].astype(jnp.float32)
             + u_ref[...])
        h = jnp.maximum(dis_ref[...] * s + b_ref[...], 0.0)
        oh = (batch_ref[...] ==
              lax.broadcasted_iota(jnp.int32, (n, NUM_GRAPHS), 1)
              ).astype(jnp.float32)
        sums = lax.dot_general(oh, h, (((0,), (0,)), ((), ())),
                               preferred_element_type=jnp.float32)
        counts = jnp.sum(oh, axis=0)
        pooled = sums / jnp.maximum(counts, 1.0)[:, None]
        out_ref[...] = jnp.dot(pooled, lw_ref[...],
                               preferred_element_type=jnp.float32) + lb_ref[...]

    return pl.pallas_call(
        body,
        out_shape=jax.ShapeDtypeStruct((NUM_GRAPHS, ncls), jnp.float32),
    )(p, u, dis, b.reshape(1, -1), batch.reshape(-1, 1), lin_w, lin_b.reshape(1, -1))


def kernel(x, edge_index, batch, W0, b0, W1, b1, W2, b2, W3, b3, W4, b4, lin_W, lin_b):
    n, d = x.shape
    e = edge_index.shape[1]
    e_per_w = e // NW
    n_chunks = e_per_w // CH
    assert n_chunks * CH == e_per_w
    src32 = edge_index[0].reshape(NW, n_chunks, CH)
    dst32 = edge_index[1].reshape(NW, n_chunks, CH)

    degp = _degree_kernel(n, n_chunks)(dst32)
    h0 = _tc_matmul(x, W0)
    dis, u, ubf = _tc_scale(degp, h0)

    scatter = _edge_scatter_kernel(n, n_chunks)
    for w_next, b_cur in ((W1, b0), (W2, b1), (W3, b2), (W4, b3)):
        p = scatter(ubf, src32, dst32)
        u, ubf = _tc_mid(p, u, dis, b_cur, w_next)

    p = scatter(ubf, src32, dst32)
    return _tc_last(p, u, dis, b4, batch, lin_W, lin_b)
